# baseline (speedup 1.0000x reference)
import functools

import jax
import jax.numpy as jnp
from jax import lax
from jax.experimental import pallas as pl
from jax.experimental.pallas import tpu as pltpu
from jax.experimental.pallas import tpu_sc as plsc

N = 10000
E = 320000
D_IN = 128
D_HID = 64
D_OUT = 128

NC = 2
NS = 16
NW = NC * NS
EPW = E // NW
K = 80
C = EPW // K
NP = 10240
RPT = NP // NS

_mesh = plsc.VectorSubcoreMesh(core_axis_name="c", subcore_axis_name="s")
_sc_params = pltpu.CompilerParams(use_tc_tiling_on_sc=False)


@functools.partial(
    pl.kernel,
    out_type=jax.ShapeDtypeStruct((NW, NP), jnp.float32),
    mesh=_mesh,
    scratch_types=[
        pltpu.VMEM((EPW,), jnp.int32),
        pltpu.VMEM((NP,), jnp.float32),
    ],
    compiler_params=pltpu.CompilerParams(
        use_tc_tiling_on_sc=False, needs_layout_passes=False
    ),
)
def _deg_sc(dst_hbm, out_hbm, dst_v, hist):
    c = lax.axis_index("c")
    s = lax.axis_index("s")
    wid = c * NS + s
    zero16 = jnp.zeros((16,), jnp.float32)
    ones16 = jnp.ones((16,), jnp.float32)

    def zb(i, carry):
        hist[pl.ds(i * 16, 16)] = zero16
        return carry

    lax.fori_loop(0, NP // 16, zb, 0)
    pltpu.sync_copy(dst_hbm.at[wid], dst_v)

    def body(r, carry):
        for q in range(5):
            ix = dst_v[pl.ds((r * 5 + q) * 16, 16)]
            plsc.addupdate_scatter(hist, [ix], ones16)
        return carry

    lax.fori_loop(0, EPW // 80, body, 0)
    pltpu.sync_copy(hist, out_hbm.at[wid])


DH = D_HID // NC
C2 = E // NS // K


@functools.partial(
    pl.kernel,
    out_type=jax.ShapeDtypeStruct((NP, D_HID), jnp.float32),
    mesh=_mesh,
    scratch_types=[
        pltpu.VMEM((C2, K), jnp.int32),
        pltpu.VMEM((C2, K), jnp.int32),
        [pltpu.VMEM((K, DH), jnp.float32) for _ in range(5)],
        pltpu.VMEM_SHARED((NP, DH), jnp.float32),
        [pltpu.SemaphoreType.DMA for _ in range(5)],
        [pltpu.SemaphoreType.DMA for _ in range(5)],
    ],
    compiler_params=_sc_params,
)
def _scatter_sc(src_hbm, dst_hbm, ya_hbm, yb_hbm, zeros_hbm, out_hbm,
                src_v, dst_v, bufs, acc, gsems, ssems):
    c = lax.axis_index("c")
    s = lax.axis_index("s")
    base = s * RPT
    pltpu.sync_copy(zeros_hbm.at[pl.ds(base, RPT)], acc.at[pl.ds(base, RPT)])
    pltpu.sync_copy(src_hbm.at[s], src_v)
    pltpu.sync_copy(dst_hbm.at[s], dst_v)
    plsc.subcore_barrier()

    U = 5

    def ring(y_hbm):
        for i in range(U):
            pltpu.async_copy(y_hbm.at[src_v.at[i]], bufs[i], gsems[i])

        def body(t, carry):
            for i in range(U):
                j = U * t + i
                pltpu.make_async_copy(
                    y_hbm.at[src_v.at[j]], bufs[i], gsems[i]).wait()
                pltpu.async_copy(bufs[i], acc.at[dst_v.at[j]], ssems[i],
                                 add=True)
            for i in range(U):
                jn = U * t + U + i
                pltpu.make_async_copy(
                    bufs[i], acc.at[dst_v.at[jn]], ssems[i]).wait()
                pltpu.async_copy(y_hbm.at[src_v.at[jn]], bufs[i], gsems[i])
            return carry

        lax.fori_loop(0, C2 // U - 1, body, 0)
        for i in range(U):
            j = C2 - U + i
            pltpu.make_async_copy(
                y_hbm.at[src_v.at[j]], bufs[i], gsems[i]).wait()
            pltpu.async_copy(bufs[i], acc.at[dst_v.at[j]], ssems[i], add=True)
        for i in range(U):
            pltpu.make_async_copy(
                bufs[i], acc.at[dst_v.at[C2 - U + i]], ssems[i]).wait()

    @pl.when(c == 0)
    def _():
        ring(ya_hbm)

    @pl.when(c == 1)
    def _():
        ring(yb_hbm)

    plsc.subcore_barrier()
    pltpu.sync_copy(acc.at[pl.ds(base, RPT)],
                    out_hbm.at[pl.ds(base, RPT), pl.ds(c * DH, DH)])


def _tc0_body(x_ref, w1_ref, xw_ref):
    xw_ref[...] = jnp.dot(
        x_ref[...], w1_ref[...], preferred_element_type=jnp.float32
    )


def _tc1_body(parts_ref, xw_ref, dinv_ref, ya_ref, yb_ref):
    deg_col = lax.dot_general(
        parts_ref[...],
        jnp.ones((NW, 1), jnp.float32),
        (((0,), (0,)), ((), ())),
        preferred_element_type=jnp.float32,
    )
    deg = deg_col[:N] + 1.0
    dinv = lax.rsqrt(deg)
    dinv_ref[...] = dinv
    y1 = dinv * xw_ref[...]
    ya_ref[...] = y1[:, :DH]
    yb_ref[...] = y1[:, DH:]


def _tc2_body(z_ref, ya_ref, yb_ref, dinv_ref, b1_ref, y2a_ref, y2b_ref):
    dinv = dinv_ref[...]
    y1 = jnp.concatenate([ya_ref[...], yb_ref[...]], axis=1)
    agg = z_ref[:N, :] + y1
    h = jnp.maximum(dinv * agg + b1_ref[...], 0.0)
    y2 = dinv * h
    y2a_ref[...] = y2[:, :DH]
    y2b_ref[...] = y2[:, DH:]


def _tc3_body(z_ref, y2a_ref, y2b_ref, dinv_ref, w2_ref, b2_ref, out_ref):
    y2 = jnp.concatenate([y2a_ref[...], y2b_ref[...]], axis=1)
    ah = dinv_ref[...] * (z_ref[:N, :] + y2)
    out_ref[...] = (
        jnp.dot(ah, w2_ref[...], preferred_element_type=jnp.float32)
        + b2_ref[...]
    )


_tc0 = pl.pallas_call(
    _tc0_body,
    out_shape=jax.ShapeDtypeStruct((N, D_HID), jnp.float32),
)
_tc1 = pl.pallas_call(
    _tc1_body,
    out_shape=(
        jax.ShapeDtypeStruct((N, 1), jnp.float32),
        jax.ShapeDtypeStruct((N, DH), jnp.float32),
        jax.ShapeDtypeStruct((N, DH), jnp.float32),
    ),
)
_tc2 = pl.pallas_call(
    _tc2_body,
    out_shape=(
        jax.ShapeDtypeStruct((N, DH), jnp.float32),
        jax.ShapeDtypeStruct((N, DH), jnp.float32),
    ),
)
_tc3 = pl.pallas_call(
    _tc3_body,
    out_shape=jax.ShapeDtypeStruct((N, D_OUT), jnp.float32),
)


def kernel(x, edge_index, W1, b1, W2, b2):
    src = edge_index[0].reshape(NS, C2, K)
    dst = edge_index[1].reshape(NS, C2, K)
    dst_flat = edge_index[1].reshape(NW, EPW)
    zeros_cols = jnp.zeros((NP, DH), jnp.float32)

    xw = _tc0(x, W1)
    deg_parts = _deg_sc(dst_flat)
    dinv, y1a, y1b = _tc1(deg_parts, xw)
    z1 = _scatter_sc(src, dst, y1a, y1b, zeros_cols)
    y2a, y2b = _tc2(z1, y1a, y1b, dinv, b1.reshape(1, D_HID))
    z2 = _scatter_sc(src, dst, y2a, y2b, zeros_cols)
    return _tc3(z2, y2a, y2b, dinv, W2, b2.reshape(1, D_OUT))


# speedup vs baseline: 1.1078x; 1.1078x over previous
import functools

import jax
import jax.numpy as jnp
from jax import lax
from jax.experimental import pallas as pl
from jax.experimental.pallas import tpu as pltpu
from jax.experimental.pallas import tpu_sc as plsc

N = 10000
E = 320000
D_IN = 128
D_HID = 64
D_OUT = 128

NC = 2
NS = 16
NW = NC * NS
EPW = E // NW
K = 80
C = EPW // K
NP = 10240
RPT = NP // NS

_mesh = plsc.VectorSubcoreMesh(core_axis_name="c", subcore_axis_name="s")
_sc_params = pltpu.CompilerParams(use_tc_tiling_on_sc=False)


@functools.partial(
    pl.kernel,
    out_type=jax.ShapeDtypeStruct((NW, NP), jnp.float32),
    mesh=_mesh,
    scratch_types=[
        pltpu.VMEM((EPW,), jnp.int32),
        pltpu.VMEM((NP,), jnp.float32),
    ],
    compiler_params=pltpu.CompilerParams(
        use_tc_tiling_on_sc=False, needs_layout_passes=False
    ),
)
def _deg_sc(dst_hbm, out_hbm, dst_v, hist):
    c = lax.axis_index("c")
    s = lax.axis_index("s")
    wid = c * NS + s
    zero16 = jnp.zeros((16,), jnp.float32)
    ones16 = jnp.ones((16,), jnp.float32)

    def zb(i, carry):
        hist[pl.ds(i * 16, 16)] = zero16
        return carry

    lax.fori_loop(0, NP // 16, zb, 0)
    pltpu.sync_copy(dst_hbm.at[wid], dst_v)

    def body(r, carry):
        for q in range(5):
            ix = dst_v[pl.ds((r * 5 + q) * 16, 16)]
            plsc.addupdate_scatter(hist, [ix], ones16)
        return carry

    lax.fori_loop(0, EPW // 80, body, 0)
    pltpu.sync_copy(hist, out_hbm.at[wid])


@functools.partial(
    pl.kernel,
    out_type=jax.ShapeDtypeStruct((NC, NP, D_HID), jnp.float32),
    mesh=_mesh,
    scratch_types=[
        pltpu.VMEM((C, K), jnp.int32),
        pltpu.VMEM((C, K), jnp.int32),
        [pltpu.VMEM((K, D_HID), jnp.float32) for _ in range(5)],
        pltpu.VMEM_SHARED((NP, D_HID), jnp.float32),
        [pltpu.SemaphoreType.DMA for _ in range(5)],
        [pltpu.SemaphoreType.DMA for _ in range(5)],
        pltpu.SemaphoreType.DMA,
    ],
    compiler_params=_sc_params,
)
def _scatter_sc(src_hbm, dst_hbm, y_hbm, zeros_hbm, out_hbm,
                src_v, dst_v, bufs, acc, gsems, ssems, psem):
    c = lax.axis_index("c")
    s = lax.axis_index("s")
    wid = c * NS + s
    base = s * RPT
    pltpu.async_copy(zeros_hbm.at[pl.ds(base, RPT)], acc.at[pl.ds(base, RPT)],
                     psem)
    pltpu.async_copy(src_hbm.at[wid], src_v, gsems[0])
    pltpu.async_copy(dst_hbm.at[wid], dst_v, gsems[1])
    pltpu.make_async_copy(src_hbm.at[wid], src_v, gsems[0]).wait()
    pltpu.make_async_copy(dst_hbm.at[wid], dst_v, gsems[1]).wait()
    pltpu.make_async_copy(
        zeros_hbm.at[pl.ds(base, RPT)], acc.at[pl.ds(base, RPT)], psem).wait()
    plsc.subcore_barrier()

    U = 5
    for i in range(U):
        pltpu.async_copy(y_hbm.at[src_v.at[i]], bufs[i], gsems[i])

    def body(t, carry):
        for i in range(U):
            j = U * t + i
            pltpu.make_async_copy(y_hbm.at[src_v.at[j]], bufs[i], gsems[i]).wait()
            pltpu.async_copy(bufs[i], acc.at[dst_v.at[j]], ssems[i], add=True)
        for i in range(U):
            jn = U * t + U + i
            pltpu.make_async_copy(bufs[i], acc.at[dst_v.at[jn]], ssems[i]).wait()
            pltpu.async_copy(y_hbm.at[src_v.at[jn]], bufs[i], gsems[i])
        return carry

    lax.fori_loop(0, C // U - 1, body, 0)
    for i in range(U):
        j = C - U + i
        pltpu.make_async_copy(y_hbm.at[src_v.at[j]], bufs[i], gsems[i]).wait()
        pltpu.async_copy(bufs[i], acc.at[dst_v.at[j]], ssems[i], add=True)
    for i in range(U):
        pltpu.make_async_copy(bufs[i], acc.at[dst_v.at[C - U + i]], ssems[i]).wait()
    plsc.subcore_barrier()
    pltpu.sync_copy(acc.at[pl.ds(base, RPT)], out_hbm.at[c, pl.ds(base, RPT)])


def _tc1_body(parts_ref, x_ref, w1_ref, dinv_ref, y1_ref):
    deg_col = lax.dot_general(
        parts_ref[...],
        jnp.ones((NW, 1), jnp.float32),
        (((0,), (0,)), ((), ())),
        preferred_element_type=jnp.float32,
    )
    deg = deg_col[:N] + 1.0
    dinv = lax.rsqrt(deg)
    dinv_ref[...] = dinv
    xw = jnp.dot(x_ref[...], w1_ref[...], preferred_element_type=jnp.float32)
    y1_ref[...] = dinv * xw


def _tc2_body(zp_ref, y1_ref, dinv_ref, b1_ref, y2_ref):
    dinv = dinv_ref[...]
    agg = zp_ref[0, :N, :] + zp_ref[1, :N, :] + y1_ref[...]
    h = jnp.maximum(dinv * agg + b1_ref[...], 0.0)
    y2_ref[...] = dinv * h


def _tc3_body(zp_ref, y2_ref, dinv_ref, w2_ref, b2_ref, out_ref):
    ah = dinv_ref[...] * (zp_ref[0, :N, :] + zp_ref[1, :N, :] + y2_ref[...])
    out_ref[...] = (
        jnp.dot(ah, w2_ref[...], preferred_element_type=jnp.float32)
        + b2_ref[...]
    )


_tc1 = pl.pallas_call(
    _tc1_body,
    out_shape=(
        jax.ShapeDtypeStruct((N, 1), jnp.float32),
        jax.ShapeDtypeStruct((N, D_HID), jnp.float32),
    ),
)
_tc2 = pl.pallas_call(
    _tc2_body,
    out_shape=jax.ShapeDtypeStruct((N, D_HID), jnp.float32),
)
_tc3 = pl.pallas_call(
    _tc3_body,
    out_shape=jax.ShapeDtypeStruct((N, D_OUT), jnp.float32),
)


def kernel(x, edge_index, W1, b1, W2, b2):
    src = edge_index[0].reshape(NW, C, K)
    dst = edge_index[1].reshape(NW, C, K)
    dst_flat = edge_index[1].reshape(NW, EPW)
    zeros_rows = jnp.zeros((NP, D_HID), jnp.float32)

    deg_parts = _deg_sc(dst_flat)
    dinv, y1 = _tc1(deg_parts, x, W1)
    z1_parts = _scatter_sc(src, dst, y1, zeros_rows)
    y2 = _tc2(z1_parts, y1, dinv, b1.reshape(1, D_HID))
    z2_parts = _scatter_sc(src, dst, y2, zeros_rows)
    return _tc3(z2_parts, y2, dinv, W2, b2.reshape(1, D_OUT))
